# Initial kernel scaffold; baseline (speedup 1.0000x reference)
import jax
import jax.numpy as jnp
from jax.experimental import pallas as pl


def kernel(q, mu, Wij, dir_ij, pairlist, W1, b1, W2, b2):
    raise NotImplementedError("write your pallas kernel here")


# speedup vs baseline: 5.7307x; 5.7307x over previous
import functools

import jax
import jax.numpy as jnp
from jax import lax
from jax.experimental import pallas as pl
from jax.experimental.pallas import tpu as pltpu
from jax.experimental.pallas import tpu_sc as plsc

F = 128
F3 = 3 * F
BK = 32
SEG = 2000
BKB = 32
GRP = 8
NP = 2


def _mlp_body(q_ref, w1_ref, b1_ref, w2_ref, b2_ref, o_ref):
    h = jnp.dot(q_ref[...], w1_ref[...], preferred_element_type=jnp.float32)
    h = h + b1_ref[...]
    h = h * jax.nn.sigmoid(h)
    o = jnp.dot(h, w2_ref[...], preferred_element_type=jnp.float32)
    o_ref[...] = o + b2_ref[...]


def _combine_body(u0_ref, u1_ref, u2_ref, u3_ref, q_ref, mu_ref,
                  qo_ref, muo_ref):
    qo_ref[...] = q_ref[...] + u0_ref[...]
    muo_ref[...] = mu_ref[...] + jnp.concatenate(
        [u1_ref[...], u2_ref[...], u3_ref[...]], axis=1)


@functools.lru_cache(maxsize=None)
def _make_sc_edges(N, E):
    info = plsc.get_sparse_core_info()
    NC, NS, L = info.num_cores, info.num_subcores, info.num_lanes
    NW = NC * NS
    EW = E // NW
    assert EW * NW == E and EW % L == 0
    NBLK = EW // BK
    TAIL = EW - NBLK * BK
    assert TAIL % L == 0

    mesh = plsc.VectorSubcoreMesh(core_axis_name="c", subcore_axis_name="s")

    @functools.partial(
        pl.kernel,
        out_type=[jax.ShapeDtypeStruct((E, F), jnp.float32)] * 4,
        mesh=mesh,
        compiler_params=pltpu.CompilerParams(needs_layout_passes=False),
        scratch_types=[
            pltpu.VMEM((EW,), jnp.int32),
            pltpu.VMEM((BK, F3), jnp.float32),
            pltpu.VMEM((BK, F3), jnp.float32),
            pltpu.VMEM((BK, F3), jnp.float32),
            pltpu.VMEM(((BK + 2) * 8,), jnp.float32),
            pltpu.VMEM((BK, F), jnp.float32),
            pltpu.VMEM((BK, F), jnp.float32),
            pltpu.VMEM((BK, F), jnp.float32),
            pltpu.VMEM((BK, F), jnp.float32),
        ],
    )
    def sc_edges(x_hbm, mu_hbm, w_hbm, d_hbm, jj_hbm,
                 u0_hbm, u1_hbm, u2_hbm, u3_hbm,
                 jjb, wbuf, xbuf, mbuf, dbuf, o0, o1, o2, o3):
        c = lax.axis_index("c")
        s = lax.axis_index("s")
        wid = s * NC + c
        ebase = wid * EW
        pltpu.sync_copy(jj_hbm.at[pl.ds(ebase, EW)], jjb)
        obufs = (o0, o1, o2, o3)
        ubufs = (u0_hbm, u1_hbm, u2_hbm, u3_hbm)

        def do_block(eb, nbk):
            pltpu.sync_copy(w_hbm.at[pl.ds(ebase + eb, nbk)],
                            wbuf.at[pl.ds(0, nbk)])
            pltpu.sync_copy(x_hbm.at[jjb.at[pl.ds(eb, nbk)]],
                            xbuf.at[pl.ds(0, nbk)])
            pltpu.sync_copy(mu_hbm.at[jjb.at[pl.ds(eb, nbk)]],
                            mbuf.at[pl.ds(0, nbk)])
            pltpu.sync_copy(d_hbm.at[pl.ds((ebase + eb) * 8, nbk * 8)],
                            dbuf.at[pl.ds(0, nbk * 8)])

            def edge(e, _):
                dv = dbuf[pl.ds(e * 8, L)]
                d0 = dv[0]
                d1 = dv[1]
                d2 = dv[2]
                for f in range(F // L):
                    sq = pl.ds(f * L, L)
                    s1 = pl.ds(F + f * L, L)
                    s2 = pl.ds(2 * F + f * L, L)
                    o0[e, sq] = wbuf[e, sq] * xbuf[e, sq]
                    t1 = wbuf[e, s1] * xbuf[e, s1]
                    t2 = wbuf[e, s2] * xbuf[e, s2]
                    o1[e, sq] = t1 * d0 + t2 * mbuf[e, sq]
                    o2[e, sq] = t1 * d1 + t2 * mbuf[e, s1]
                    o3[e, sq] = t1 * d2 + t2 * mbuf[e, s2]
                return 0

            lax.fori_loop(0, nbk, edge, 0)
            for qq in range(4):
                pltpu.sync_copy(obufs[qq].at[pl.ds(0, nbk)],
                                ubufs[qq].at[pl.ds(ebase + eb, nbk)])

        def blk(k, _):
            do_block(k * BK, BK)
            return 0

        lax.fori_loop(0, NBLK, blk, 0)
        if TAIL:
            do_block(NBLK * BK, TAIL)

    return sc_edges


@functools.lru_cache(maxsize=None)
def _make_sc_reduce(N, E):
    info = plsc.get_sparse_core_info()
    NC, NS, L = info.num_cores, info.num_subcores, info.num_lanes
    NW = NC * NS
    assert NW == GRP * 4
    BA = N // (GRP * NP)
    assert BA * GRP * NP == N
    ACCR = ((BA + 1 + 7) // 8) * 8
    NSEG = E // SEG
    assert NSEG * SEG == E and SEG % L == 0
    LSZ = SEG + BKB

    mesh = plsc.VectorSubcoreMesh(core_axis_name="c", subcore_axis_name="s")

    @functools.partial(
        pl.kernel,
        out_type=[jax.ShapeDtypeStruct((N * F,), jnp.float32)] * 4,
        mesh=mesh,
        compiler_params=pltpu.CompilerParams(needs_layout_passes=False),
        scratch_types=[
            pltpu.VMEM((SEG,), jnp.int32),
            pltpu.VMEM((LSZ,), jnp.int32),
            pltpu.VMEM((LSZ,), jnp.int32),
            pltpu.VMEM((BKB, F), jnp.float32),
            pltpu.VMEM((ACCR * F,), jnp.float32),
        ],
    )
    def sc_reduce(ii_hbm, u0_hbm, u1_hbm, u2_hbm, u3_hbm,
                  v0_hbm, v1_hbm, v2_hbm, v3_hbm,
                  iibuf, ls_e, ls_r, gbuf, acc):
        c = lax.axis_index("c")
        s = lax.axis_index("s")
        wid = s * NC + c
        g = wid // 4
        q = wid % 4
        ubufs = (u0_hbm, u1_hbm, u2_hbm, u3_hbm)
        vbufs = (v0_hbm, v1_hbm, v2_hbm, v3_hbm)
        lane = lax.iota(jnp.int32, L)
        fconst = [f * L + lane for f in range(F // L)]
        zf = jnp.zeros((L,), jnp.float32)

        for p in range(NP):
            base = (p * GRP + g) * BA

            def zero(i, _):
                acc[pl.ds(i * L, L)] = zf
                return 0

            lax.fori_loop(0, ACCR * F // L, zero, 0)

            def seg_fn(sg, _):
                pltpu.sync_copy(ii_hbm.at[pl.ds(sg * SEG, SEG)], iibuf)

                def build(t, cnt):
                    iv = iibuf[pl.ds(t * L, L)]
                    m = jnp.logical_and(iv >= base, iv < base + BA)
                    mi = m.astype(jnp.int32)
                    pos = cnt + plsc.cumsum(mi) - 1
                    eid = sg * SEG + t * L + lane
                    plsc.store_scatter(ls_e, [pos], eid, mask=m)
                    plsc.store_scatter(ls_r, [pos], (iv - base) * F, mask=m)
                    return cnt + jnp.sum(mi)

                cnt = lax.fori_loop(0, SEG // L, build, jnp.int32(0))
                for h in range(BKB // L):
                    padp = cnt + h * L + lane
                    plsc.store_scatter(ls_e, [padp], lane)
                    plsc.store_scatter(ls_r, [padp],
                                       jnp.full((L,), BA * F, jnp.int32))
                nblk = (cnt + BKB - 1) // BKB

                def blk(k, _):
                    eb = k * BKB
                    for qq in range(4):
                        @pl.when(q == qq)
                        def _():
                            pltpu.sync_copy(
                                ubufs[qq].at[ls_e.at[pl.ds(eb, BKB)]], gbuf)
                    for h in range(BKB // L):
                        rv = ls_r[pl.ds(eb + h * L, L)]

                        def edge(e, _):
                            ei = jnp.full((L,), e, jnp.int32)
                            radr = rv.at[ei].get(mode="promise_in_bounds")
                            for f in range(F // L):
                                vals = gbuf[h * L + e, pl.ds(f * L, L)]
                                plsc.addupdate_scatter(
                                    acc, [radr + fconst[f]], vals)
                            return 0

                        lax.fori_loop(0, L, edge, 0)
                    return 0

                lax.fori_loop(0, nblk, blk, 0)
                return 0

            lax.fori_loop(0, NSEG, seg_fn, 0)

            for qq in range(4):
                @pl.when(q == qq)
                def _():
                    pltpu.sync_copy(acc.at[pl.ds(0, BA * F)],
                                    vbufs[qq].at[pl.ds(base * F, BA * F)])

    return sc_reduce


def kernel(q, mu, Wij, dir_ij, pairlist, W1, b1, W2, b2):
    N = q.shape[0]
    E = Wij.shape[0]
    q2 = q.reshape(N, F)
    mu2 = mu.reshape(N, F3)
    wij2 = Wij.reshape(E, F3)
    dirp = jnp.pad(dir_ij, ((0, 0), (0, 5))).reshape(E * 8)
    idx_i = pairlist[0].astype(jnp.int32)
    idx_j = pairlist[1].astype(jnp.int32)

    BN = 1000
    x_tab = pl.pallas_call(
        _mlp_body,
        grid=(N // BN,),
        in_specs=[
            pl.BlockSpec((BN, F), lambda i: (i, 0)),
            pl.BlockSpec((F, F), lambda i: (0, 0)),
            pl.BlockSpec((1, F), lambda i: (0, 0)),
            pl.BlockSpec((F, F3), lambda i: (0, 0)),
            pl.BlockSpec((1, F3), lambda i: (0, 0)),
        ],
        out_specs=pl.BlockSpec((BN, F3), lambda i: (i, 0)),
        out_shape=jax.ShapeDtypeStruct((N, F3), jnp.float32),
    )(q2, W1, b1.reshape(1, F), W2, b2.reshape(1, F3))

    U = _make_sc_edges(N, E)(x_tab, mu2, wij2, dirp, idx_j)
    V = _make_sc_reduce(N, E)(idx_i, *U)
    V = [v.reshape(N, F) for v in V]

    q_out, mu_out = pl.pallas_call(
        _combine_body,
        grid=(N // BN,),
        in_specs=[pl.BlockSpec((BN, F), lambda i: (i, 0))] * 4 + [
            pl.BlockSpec((BN, F), lambda i: (i, 0)),
            pl.BlockSpec((BN, F3), lambda i: (i, 0)),
        ],
        out_specs=[
            pl.BlockSpec((BN, F), lambda i: (i, 0)),
            pl.BlockSpec((BN, F3), lambda i: (i, 0)),
        ],
        out_shape=[
            jax.ShapeDtypeStruct((N, F), jnp.float32),
            jax.ShapeDtypeStruct((N, F3), jnp.float32),
        ],
    )(*V, q2, mu2)

    return (q_out.reshape(N, 1, F), mu_out.reshape(N, 3, F))
